# baseline (speedup 1.0000x reference)
import functools

import jax
import jax.numpy as jnp
from jax import lax
from jax.experimental import pallas as pl
from jax.experimental.pallas import tpu as pltpu
from jax.experimental.pallas import tpu_sc as plsc

BATCH = 16384
D = 32
NC = 2
NS = 16
NW = NC * NS
BPW = BATCH // NW
CHUNK = 128
NCHUNK = BPW // CHUNK
L = 16


def _mf_body(user_hbm, item_hbm, mu_hbm, ub_hbm, ib_hbm, ufac_hbm, ifac_hbm,
             out_hbm, uidx, iidx, uf, itf, ub, ib, outv, muv, sem):
    wid = lax.axis_index("s") * NC + lax.axis_index("c")
    base_row = wid * NCHUNK

    pltpu.sync_copy(user_hbm.at[pl.ds(base_row, NCHUNK)], uidx)
    pltpu.sync_copy(item_hbm.at[pl.ds(base_row, NCHUNK)], iidx)
    pltpu.sync_copy(mu_hbm, muv)

    copies = []
    for j in range(NCHUNK):
        copies.append(pltpu.async_copy(
            ufac_hbm.at[uidx.at[j]], uf.at[pl.ds(j * CHUNK, CHUNK)], sem))
        copies.append(pltpu.async_copy(
            ifac_hbm.at[iidx.at[j]], itf.at[pl.ds(j * CHUNK, CHUNK)], sem))
        copies.append(pltpu.async_copy(
            ub_hbm.at[uidx.at[j]], ub.at[pl.ds(j * CHUNK, CHUNK)], sem))
        copies.append(pltpu.async_copy(
            ib_hbm.at[iidx.at[j]], ib.at[pl.ds(j * CHUNK, CHUNK)], sem))
    for c in copies:
        c.wait()

    mu_v = muv[...]

    def tbody(t, carry):
        r = t * L + lax.iota(jnp.int32, L)
        acc = ub[pl.ds(t * L, L)] + ib[pl.ds(t * L, L)] + mu_v
        for f in range(D):
            fv = jnp.full((L,), f, jnp.int32)
            acc = acc + (plsc.load_gather(uf, [r, fv])
                         * plsc.load_gather(itf, [r, fv]))
        plsc.store_scatter(outv, [r], acc)
        return carry

    lax.fori_loop(0, BPW // L, tbody, 0)

    pltpu.sync_copy(outv, out_hbm.at[pl.ds(wid * BPW, BPW)])


@functools.partial(jax.jit, donate_argnums=())
def _mf(user2, item2, mu, ub2, ib2, ufac, ifac):
    mesh = plsc.VectorSubcoreMesh(core_axis_name="c", subcore_axis_name="s")
    f = pl.kernel(
        _mf_body,
        out_type=jax.ShapeDtypeStruct((BATCH,), jnp.float32),
        mesh=mesh,
        compiler_params=pltpu.CompilerParams(
            needs_layout_passes=False, use_tc_tiling_on_sc=False),
        scratch_types=[
            pltpu.VMEM((NCHUNK, CHUNK), jnp.int32),
            pltpu.VMEM((NCHUNK, CHUNK), jnp.int32),
            pltpu.VMEM((BPW, D), jnp.float32),
            pltpu.VMEM((BPW, D), jnp.float32),
            pltpu.VMEM((BPW,), jnp.float32),
            pltpu.VMEM((BPW,), jnp.float32),
            pltpu.VMEM((BPW,), jnp.float32),
            pltpu.VMEM((L,), jnp.float32),
            pltpu.SemaphoreType.DMA,
        ],
    )
    return f(user2, item2, mu, ub2, ib2, ufac, ifac)


def kernel(user, item, mu, user_bias, item_bias, user_factors, item_factors):
    user2 = user.reshape(NW * NCHUNK, CHUNK)
    item2 = item.reshape(NW * NCHUNK, CHUNK)
    mu16 = jnp.broadcast_to(mu, (L,))
    return _mf(user2, item2, mu16, user_bias, item_bias,
               user_factors, item_factors)


# speedup vs baseline: 1.0006x; 1.0006x over previous
import functools

import jax
import jax.numpy as jnp
from jax import lax
from jax.experimental import pallas as pl
from jax.experimental.pallas import tpu as pltpu
from jax.experimental.pallas import tpu_sc as plsc

BATCH = 16384
D = 32
PACK = 4
NC = 2
NS = 16
NW = NC * NS
BPW = BATCH // NW
CHUNK = 128
NCHUNK = BPW // CHUNK
L = 16
G = CHUNK // L


def _mf_body(user_hbm, item_hbm, mu_hbm, ub_hbm, ib_hbm, ufac_hbm, ifac_hbm,
             out_hbm, uidx, iidx, umac, imac, ubuf, ibuf, ub, ib, outv, muv,
             sem0, sem1):
    wid = lax.axis_index("s") * NC + lax.axis_index("c")
    base_row = wid * NCHUNK
    sems = (sem0, sem1)

    pltpu.sync_copy(user_hbm.at[pl.ds(base_row, NCHUNK)], uidx)
    pltpu.sync_copy(item_hbm.at[pl.ds(base_row, NCHUNK)], iidx)
    pltpu.sync_copy(mu_hbm, muv)

    for j in range(NCHUNK):
        for g in range(G):
            s = pl.ds(g * L, L)
            umac[j, s] = jnp.right_shift(uidx[j, s], 2)
            imac[j, s] = jnp.right_shift(iidx[j, s], 2)

    def start(j):
        slot = j % 2
        cu = pltpu.async_copy(ufac_hbm.at[umac.at[j]], ubuf.at[slot],
                              sems[slot])
        ci = pltpu.async_copy(ifac_hbm.at[imac.at[j]], ibuf.at[slot],
                              sems[slot])
        cb = pltpu.async_copy(ub_hbm.at[uidx.at[j]],
                              ub.at[pl.ds(j * CHUNK, CHUNK)], sems[slot])
        db = pltpu.async_copy(ib_hbm.at[iidx.at[j]],
                              ib.at[pl.ds(j * CHUNK, CHUNK)], sems[slot])
        return (cu, ci, cb, db)

    mu_v = muv[...]
    inflight = start(0)
    for j in range(NCHUNK):
        for c in inflight:
            c.wait()
        if j + 1 < NCHUNK:
            nxt = start(j + 1)
        slot = j % 2
        ubj = ubuf.at[slot]
        ibj = ibuf.at[slot]

        def gbody(g, carry):
            rv = g * L + lax.iota(jnp.int32, L)
            s = pl.ds(g * L, L)
            cu = jnp.left_shift(jnp.bitwise_and(uidx[j, s], PACK - 1), 5)
            ci = jnp.left_shift(jnp.bitwise_and(iidx[j, s], PACK - 1), 5)
            o = pl.ds(j * CHUNK + g * L, L)
            acc = ub[o] + ib[o] + mu_v
            for f in range(D):
                acc = acc + (plsc.load_gather(ubj, [rv, cu + f])
                             * plsc.load_gather(ibj, [rv, ci + f]))
            outv[o] = acc
            return carry

        lax.fori_loop(0, G, gbody, 0)
        if j + 1 < NCHUNK:
            inflight = nxt

    pltpu.sync_copy(outv, out_hbm.at[pl.ds(wid * BPW, BPW)])


@jax.jit
def _mf(user2, item2, mu, ub_t, ib_t, ufac4, ifac4):
    mesh = plsc.VectorSubcoreMesh(core_axis_name="c", subcore_axis_name="s")
    f = pl.kernel(
        _mf_body,
        out_type=jax.ShapeDtypeStruct((BATCH,), jnp.float32),
        mesh=mesh,
        compiler_params=pltpu.CompilerParams(
            needs_layout_passes=False, use_tc_tiling_on_sc=False),
        scratch_types=[
            pltpu.VMEM((NCHUNK, CHUNK), jnp.int32),
            pltpu.VMEM((NCHUNK, CHUNK), jnp.int32),
            pltpu.VMEM((NCHUNK, CHUNK), jnp.int32),
            pltpu.VMEM((NCHUNK, CHUNK), jnp.int32),
            pltpu.VMEM((2, CHUNK, PACK * D), jnp.float32),
            pltpu.VMEM((2, CHUNK, PACK * D), jnp.float32),
            pltpu.VMEM((BPW,), jnp.float32),
            pltpu.VMEM((BPW,), jnp.float32),
            pltpu.VMEM((BPW,), jnp.float32),
            pltpu.VMEM((L,), jnp.float32),
            pltpu.SemaphoreType.DMA,
            pltpu.SemaphoreType.DMA,
        ],
    )
    return f(user2, item2, mu, ub_t, ib_t, ufac4, ifac4)


def kernel(user, item, mu, user_bias, item_bias, user_factors, item_factors):
    user2 = user.reshape(NW * NCHUNK, CHUNK)
    item2 = item.reshape(NW * NCHUNK, CHUNK)
    ufac4 = user_factors.reshape(-1, PACK * D)
    ifac4 = item_factors.reshape(-1, PACK * D)
    mu16 = jnp.broadcast_to(mu, (L,))
    return _mf(user2, item2, mu16, user_bias, item_bias, ufac4, ifac4)
